# Initial kernel scaffold; baseline (speedup 1.0000x reference)
import jax
import jax.numpy as jnp
from jax.experimental import pallas as pl


def kernel(adj_logits, gumbel_noise):
    raise NotImplementedError("write your pallas kernel here")


# speedup vs baseline: 4.5788x; 4.5788x over previous
import functools

import jax
import jax.numpy as jnp
import numpy as np
from jax import lax
from jax.experimental import pallas as pl
from jax.experimental.pallas import tpu as pltpu
from jax.experimental.pallas import tpu_sc as plsc

_M = 512
_PAIRS = _M * (_M - 1) // 2
_ROWS = 2 * _PAIRS // _M
_WORDS = _PAIRS // 2
_NG = 16
_NC = 4


def _pack_matrix() -> np.ndarray:
    p = np.zeros((_M, 128), dtype=np.float32)
    w = np.arange(128)
    p[4 * w, w] = 1.0
    p[4 * w + 2, w] = 65536.0
    return p


def _phase1_body(a_ref, b_ref, p_ref, o_ref, acc_ref):
    c = pl.program_id(1)
    s = a_ref[0] + b_ref[0]
    t = jnp.roll(s, -1, axis=1)
    x = (s >= t).astype(jnp.float32)

    @pl.when(c == 0)
    def _():
        acc_ref[...] = x

    @pl.when(c != 0)
    def _():
        acc_ref[...] = acc_ref[...] + x

    @pl.when(c == _NC - 1)
    def _():
        packed = jnp.dot(acc_ref[...].astype(jnp.bfloat16), p_ref[...],
                         preferred_element_type=jnp.float32)
        o_ref[0] = packed.astype(jnp.int32)


def _phase1(a, b, pmat):
    return pl.pallas_call(
        _phase1_body,
        grid=(_NG, _NC),
        in_specs=[
            pl.BlockSpec((1, _ROWS, _M), lambda g, c: (_NC * g + c, 0, 0)),
            pl.BlockSpec((1, _ROWS, _M), lambda g, c: (_NC * g + c, 0, 0)),
            pl.BlockSpec((_M, 128), lambda g, c: (0, 0)),
        ],
        out_specs=pl.BlockSpec((1, _ROWS, 128), lambda g, c: (g, 0, 0)),
        out_shape=jax.ShapeDtypeStruct((_NG, _ROWS, 128), jnp.int32),
        scratch_shapes=[pltpu.VMEM((_ROWS, _M), jnp.float32)],
    )(a, b, pmat)


_SC_MESH = plsc.VectorSubcoreMesh(core_axis_name="c", subcore_axis_name="s")
_BATCH = 8


@functools.partial(
    pl.kernel,
    mesh=_SC_MESH,
    out_type=jax.ShapeDtypeStruct((_NG, _M, _M), jnp.float32),
    scratch_types=[
        pltpu.VMEM((_WORDS,), jnp.int32),
        pltpu.VMEM((_BATCH, _M), jnp.float32),
    ],
    compiler_params=pltpu.CompilerParams(needs_layout_passes=False),
)
def _sc_expand(xsp_hbm, out_hbm, plane_v, rows_v):
    g = lax.axis_index("s")
    h = lax.axis_index("c")
    pltpu.sync_copy(xsp_hbm.at[g], plane_v)
    nn = 512 - 32 * lax.rem(g, 8)
    base_row = h * 256
    jot = lax.iota(jnp.int32, 16)

    def batch_body(bidx, _):
        i0 = base_row + bidx * _BATCH
        for r in range(_BATCH):
            i = i0 + r
            offi = i * 511 - ((i * (i - 1)) >> 1)
            upper_c = offi - i - 1
            rowf = jnp.where(i < nn, 1.0, 0.0)

            def chunk_body(k, _):
                j = jot + k * 16
                offj = j * 511 - ((j * (j - 1)) >> 1)
                idx = jnp.where(j > i, upper_c + j, offj + (i - 1) - j)
                idx = jnp.maximum(idx, 0)
                w = plsc.load_gather(plane_v, [lax.shift_right_logical(idx, 1)])
                v = lax.shift_right_logical(w, (idx & 1) << 4) & 0xFFFF
                m = jnp.where((j < nn) & (j != i), rowf, 0.0)
                rows_v[r, pl.ds(k * 16, 16)] = v.astype(jnp.float32) * m
                return 0

            lax.fori_loop(0, 32, chunk_body, 0)
        pltpu.sync_copy(rows_v, out_hbm.at[g, pl.ds(i0, _BATCH)])
        return 0

    lax.fori_loop(0, 256 // _BATCH, batch_body, 0)


def kernel(adj_logits, gumbel_noise):
    a = adj_logits.reshape(_NG * _NC, _ROWS, _M)
    b = gumbel_noise.reshape(_NG * _NC, _ROWS, _M)
    pmat = jnp.asarray(_pack_matrix(), dtype=jnp.bfloat16)
    xsp = _phase1(a, b, pmat)
    return _sc_expand(xsp.reshape(_NG, _WORDS))


# speedup vs baseline: 6.2284x; 1.3603x over previous
import functools

import jax
import jax.numpy as jnp
import numpy as np
from jax import lax
from jax.experimental import pallas as pl
from jax.experimental.pallas import tpu as pltpu
from jax.experimental.pallas import tpu_sc as plsc

_M = 512
_PAIRS = _M * (_M - 1) // 2
_RPC = 511
_BLK = 8 * _RPC
_NG = 16
_NC = 4


def _sum_matrix() -> np.ndarray:
    l = np.zeros((1024, _BLK), dtype=np.float32)
    r = np.arange(_RPC)
    for gamma in range(2):
        for c in range(_NC):
            l[gamma * 512 + r, (4 * gamma + c) * _RPC + r] = 1.0
    return l


def _pack_matrix() -> np.ndarray:
    p = np.zeros((256, 128), dtype=np.float32)
    u = np.arange(128)
    p[2 * u, u] = 1.0
    p[2 * u + 1, u] = 65536.0
    return p


_SUB = 584


def _phase1_body(a_ref, b_ref, l_ref, p_ref, o_ref, x_ref):
    d = pl.program_id(1)
    s = a_ref[...] + b_ref[...]
    x_lo = (s[:, 0:128] >= s[:, 128:256])
    x_hi = (s[:, 256:384] >= s[:, 384:512])
    x = jnp.concatenate([x_lo, x_hi], axis=1).astype(jnp.bfloat16)
    x_ref[pl.ds(d * _SUB, _SUB), :] = x

    @pl.when(d == 6)
    def _():
        y = jnp.dot(l_ref[...], x_ref[...],
                    preferred_element_type=jnp.float32)
        packed = jnp.dot(y.astype(jnp.bfloat16), p_ref[...],
                         preferred_element_type=jnp.float32)
        o_ref[...] = packed.astype(jnp.int32)


def _phase1(a2, b2, lmat, pmat):
    return pl.pallas_call(
        _phase1_body,
        grid=(8, 7),
        in_specs=[
            pl.BlockSpec((_SUB, 512), lambda b, d: (7 * b + d, 0)),
            pl.BlockSpec((_SUB, 512), lambda b, d: (7 * b + d, 0)),
            pl.BlockSpec((1024, _BLK), lambda b, d: (0, 0)),
            pl.BlockSpec((256, 128), lambda b, d: (0, 0)),
        ],
        out_specs=pl.BlockSpec((1024, 128), lambda b, d: (b, 0)),
        out_shape=jax.ShapeDtypeStruct((8192, 128), jnp.int32),
        scratch_shapes=[pltpu.VMEM((_BLK, 256), jnp.bfloat16)],
    )(a2, b2, lmat, pmat)


_BATCH = 8


@functools.cache
def _sc_expand_fn():
    mesh = plsc.VectorSubcoreMesh(core_axis_name="c", subcore_axis_name="s")
    return pl.kernel(
        _sc_expand,
        mesh=mesh,
        out_type=jax.ShapeDtypeStruct((_NG, _M, _M), jnp.float32),
        scratch_types=[
            pltpu.VMEM((512, 128), jnp.int32),
            pltpu.VMEM((_BATCH, _M), jnp.float32),
        ],
        compiler_params=pltpu.CompilerParams(needs_layout_passes=False),
    )


def _sc_expand(xsp_hbm, out_hbm, plane_v, rows_v):
    g = lax.axis_index("s")
    h = lax.axis_index("c")
    pltpu.sync_copy(xsp_hbm.at[g], plane_v)
    nn = 512 - 32 * lax.rem(g, 8)
    base_row = h * 256
    jot = lax.iota(jnp.int32, 16)

    def batch_body(bidx, _):
        i0 = base_row + bidx * _BATCH
        for r in range(_BATCH):
            i = i0 + r
            offi = i * 511 - ((i * (i - 1)) >> 1)
            upper_c = offi - i - 1
            rowf = jnp.where(i < nn, 1.0, 0.0)

            def chunk_body(k, _):
                j = jot + k * 16
                offj = j * 511 - ((j * (j - 1)) >> 1)
                idx = jnp.where(j > i, upper_c + j, offj + (i - 1) - j)
                idx = jnp.maximum(idx, 0)
                w_idx = lax.shift_right_logical(idx, 1)
                w = plsc.load_gather(
                    plane_v, [lax.shift_right_logical(w_idx, 7), w_idx & 127])
                v = lax.shift_right_logical(w, (idx & 1) << 4) & 0xFFFF
                m = jnp.where((j < nn) & (j != i), rowf, 0.0)
                rows_v[r, pl.ds(k * 16, 16)] = v.astype(jnp.float32) * m
                return 0

            lax.fori_loop(0, 32, chunk_body, 0)
        pltpu.sync_copy(rows_v, out_hbm.at[g, pl.ds(i0, _BATCH)])
        return 0

    lax.fori_loop(0, 256 // _BATCH, batch_body, 0)


def _as_rows(x):
    return (x.reshape(64, _RPC, 2, 128, 2)
             .transpose(0, 1, 2, 4, 3)
             .reshape(64 * _RPC, 512))


def kernel(adj_logits, gumbel_noise):
    a2 = _as_rows(adj_logits)
    b2 = _as_rows(gumbel_noise)
    lmat = jnp.asarray(_sum_matrix(), dtype=jnp.bfloat16)
    pmat = jnp.asarray(_pack_matrix(), dtype=jnp.bfloat16)
    xsp = _phase1(a2, b2, lmat, pmat)
    return _sc_expand_fn()(xsp.reshape(_NG, _M, 128))


# speedup vs baseline: 9.7730x; 1.5691x over previous
import functools

import jax
import jax.numpy as jnp
import numpy as np
from jax import lax
from jax.experimental import pallas as pl
from jax.experimental.pallas import tpu as pltpu
from jax.experimental.pallas import tpu_sc as plsc

_M = 512
_PAIRS = _M * (_M - 1) // 2
_RPC = 511
_BLK = 8 * _RPC
_NG = 16
_NC = 4


def _merge_matrices() -> tuple[np.ndarray, np.ndarray]:
    la = np.zeros((512, 2044), dtype=np.float32)
    lb = np.zeros((512, 2044), dtype=np.float32)
    r = np.arange(511)
    la[r, 4 * r] = 1.0
    lb[r, 4 * r + 2] = 1.0
    return la, lb


def _pack_matrix() -> np.ndarray:
    p = np.zeros((128, 64), dtype=np.float32)
    u = np.arange(64)
    p[2 * u, u] = 1.0
    p[2 * u + 1, u] = 65536.0
    return p


def _phase1_body(a_ref, b_ref, la_ref, lb_ref, p_ref, o_ref):
    s = a_ref[...] + b_ref[...]
    t = jnp.roll(s, 1, axis=2)
    xf = (s >= t).astype(jnp.bfloat16)
    for gamma in range(2):
        g4 = 4 * gamma
        xs = xf[g4] + xf[g4 + 1] + xf[g4 + 2] + xf[g4 + 3]
        x2 = xs.reshape(2044, 128)
        ya = jnp.dot(la_ref[...], x2, preferred_element_type=jnp.float32)
        yb = jnp.dot(lb_ref[...], x2, preferred_element_type=jnp.float32)
        pa = jnp.dot(ya.astype(jnp.bfloat16), p_ref[...],
                     preferred_element_type=jnp.float32)
        pb = jnp.dot(yb.astype(jnp.bfloat16), p_ref[...],
                     preferred_element_type=jnp.float32)
        packed = jnp.concatenate([pa, pb], axis=1).astype(jnp.int32)
        o_ref[pl.ds(gamma * 512, 512), :] = packed


def _phase1(a4, b4, la, lb, pmat):
    return pl.pallas_call(
        _phase1_body,
        grid=(8,),
        in_specs=[
            pl.BlockSpec((8, 1022, 2, 128), lambda b: (b, 0, 0, 0)),
            pl.BlockSpec((8, 1022, 2, 128), lambda b: (b, 0, 0, 0)),
            pl.BlockSpec((512, 2044), lambda b: (0, 0)),
            pl.BlockSpec((512, 2044), lambda b: (0, 0)),
            pl.BlockSpec((128, 64), lambda b: (0, 0)),
        ],
        out_specs=pl.BlockSpec((1024, 128), lambda b: (b, 0)),
        out_shape=jax.ShapeDtypeStruct((8192, 128), jnp.int32),
    )(a4, b4, la, lb, pmat)


_BATCH = 8


@functools.cache
def _sc_expand_fn():
    mesh = plsc.VectorSubcoreMesh(core_axis_name="c", subcore_axis_name="s")
    return pl.kernel(
        _sc_expand,
        mesh=mesh,
        out_type=jax.ShapeDtypeStruct((_NG, _M, _M), jnp.float32),
        scratch_types=[
            pltpu.VMEM((512, 128), jnp.int32),
            pltpu.VMEM((_BATCH, _M), jnp.float32),
        ],
        compiler_params=pltpu.CompilerParams(needs_layout_passes=False),
    )


def _sc_expand(xsp_hbm, out_hbm, plane_v, rows_v):
    g = lax.axis_index("s")
    h = lax.axis_index("c")
    pltpu.sync_copy(xsp_hbm.at[g], plane_v)
    nn = 512 - 32 * lax.rem(g, 8)
    base_row = h * 256
    jot = lax.iota(jnp.int32, 16)

    def batch_body(bidx, _):
        i0 = base_row + bidx * _BATCH
        for r in range(_BATCH):
            i = i0 + r
            offi = i * 511 - ((i * (i - 1)) >> 1)
            upper_c = offi - i - 1
            rowf = jnp.where(i < nn, 1.0, 0.0)

            def chunk_body(k, _):
                j = jot + k * 16
                offj = j * 511 - ((j * (j - 1)) >> 1)
                idx = jnp.where(j > i, upper_c + j, offj + (i - 1) - j)
                idx = jnp.maximum(idx, 0)
                w_idx = lax.shift_right_logical(idx, 1)
                w = plsc.load_gather(
                    plane_v, [lax.shift_right_logical(w_idx, 7), w_idx & 127])
                v = lax.shift_right_logical(w, (idx & 1) << 4) & 0xFFFF
                m = jnp.where((j < nn) & (j != i), rowf, 0.0)
                rows_v[r, pl.ds(k * 16, 16)] = v.astype(jnp.float32) * m
                return 0

            lax.fori_loop(0, 32, chunk_body, 0)
        pltpu.sync_copy(rows_v, out_hbm.at[g, pl.ds(i0, _BATCH)])
        return 0

    lax.fori_loop(0, 256 // _BATCH, batch_body, 0)


def _as_tiles(x):
    return x.reshape(64, 1022, 128, 2).transpose(0, 1, 3, 2)


def kernel(adj_logits, gumbel_noise):
    a4 = _as_tiles(adj_logits)
    b4 = _as_tiles(gumbel_noise)
    la, lb = _merge_matrices()
    la = jnp.asarray(la, dtype=jnp.bfloat16)
    lb = jnp.asarray(lb, dtype=jnp.bfloat16)
    pmat = jnp.asarray(_pack_matrix(), dtype=jnp.bfloat16)
    xsp = _phase1(a4, b4, la, lb, pmat)
    return _sc_expand_fn()(xsp.reshape(_NG, _M, 128))


# speedup vs baseline: 15.3785x; 1.5736x over previous
import functools

import jax
import jax.numpy as jnp
import numpy as np
from jax import lax
from jax.experimental import pallas as pl
from jax.experimental.pallas import tpu as pltpu
from jax.experimental.pallas import tpu_sc as plsc

_M = 512
_PAIRS = _M * (_M - 1) // 2
_RPC = 511
_BLK = 8 * _RPC
_NG = 16
_NC = 4


def _merge_matrices() -> tuple[np.ndarray, np.ndarray]:
    la = np.zeros((512, 1022), dtype=np.float32)
    lb = np.zeros((512, 1022), dtype=np.float32)
    r = np.arange(511)
    la[r, 2 * r] = 1.0
    lb[r, 2 * r + 1] = 1.0
    return la, lb


def _pack_matrix() -> np.ndarray:
    p = np.zeros((128, 64), dtype=np.float32)
    u = np.arange(64)
    p[2 * u, u] = 1.0
    p[2 * u + 1, u] = 65536.0
    return p


def _start_copies(a_hbm, b_hbm, bufs, sems, step, slot):
    for ch in range(2):
        pltpu.make_async_copy(
            a_hbm.at[pl.ds(step * 8, 8), :, ch, 0, :],
            bufs[ch].at[slot], sems.at[slot, ch]).start()
        pltpu.make_async_copy(
            b_hbm.at[pl.ds(step * 8, 8), :, ch, 0, :],
            bufs[2 + ch].at[slot], sems.at[slot, 2 + ch]).start()


def _wait_copies(a_hbm, b_hbm, bufs, sems, step, slot):
    for ch in range(2):
        pltpu.make_async_copy(
            a_hbm.at[pl.ds(step * 8, 8), :, ch, 0, :],
            bufs[ch].at[slot], sems.at[slot, ch]).wait()
        pltpu.make_async_copy(
            b_hbm.at[pl.ds(step * 8, 8), :, ch, 0, :],
            bufs[2 + ch].at[slot], sems.at[slot, 2 + ch]).wait()


def _phase1_body(a_hbm, b_hbm, la_ref, lb_ref, p_ref, o_ref,
                 a0b, a1b, b0b, b1b, sems):
    b = pl.program_id(0)
    slot = lax.rem(b, 2)
    bufs = (a0b, a1b, b0b, b1b)

    @pl.when(b == 0)
    def _():
        _start_copies(a_hbm, b_hbm, bufs, sems, 0, 0)

    @pl.when(b < 7)
    def _():
        _start_copies(a_hbm, b_hbm, bufs, sems, b + 1, 1 - slot)

    _wait_copies(a_hbm, b_hbm, bufs, sems, b, slot)

    s0 = a0b[slot] + b0b[slot]
    s1 = a1b[slot] + b1b[slot]
    x = (s0 >= s1).astype(jnp.bfloat16)
    for gamma in range(2):
        g4 = 4 * gamma
        xs = x[g4] + x[g4 + 1] + x[g4 + 2] + x[g4 + 3]
        ya = jnp.dot(la_ref[...], xs, preferred_element_type=jnp.float32)
        yb = jnp.dot(lb_ref[...], xs, preferred_element_type=jnp.float32)
        pa = jnp.dot(ya.astype(jnp.bfloat16), p_ref[...],
                     preferred_element_type=jnp.float32)
        pb = jnp.dot(yb.astype(jnp.bfloat16), p_ref[...],
                     preferred_element_type=jnp.float32)
        packed = jnp.concatenate([pa, pb], axis=1).astype(jnp.int32)
        o_ref[pl.ds(gamma * 512, 512), :] = packed


def _phase1(a5, b5, la, lb, pmat):
    return pl.pallas_call(
        _phase1_body,
        grid=(8,),
        in_specs=[
            pl.BlockSpec(memory_space=pl.ANY),
            pl.BlockSpec(memory_space=pl.ANY),
            pl.BlockSpec((512, 1022), lambda b: (0, 0)),
            pl.BlockSpec((512, 1022), lambda b: (0, 0)),
            pl.BlockSpec((128, 64), lambda b: (0, 0)),
        ],
        out_specs=pl.BlockSpec((1024, 128), lambda b: (b, 0)),
        out_shape=jax.ShapeDtypeStruct((8192, 128), jnp.int32),
        scratch_shapes=[
            pltpu.VMEM((2, 8, 1022, 128), jnp.float32),
            pltpu.VMEM((2, 8, 1022, 128), jnp.float32),
            pltpu.VMEM((2, 8, 1022, 128), jnp.float32),
            pltpu.VMEM((2, 8, 1022, 128), jnp.float32),
            pltpu.SemaphoreType.DMA((2, 4)),
        ],
    )(a5, b5, la, lb, pmat)


_BATCH = 8


@functools.cache
def _sc_expand_fn():
    mesh = plsc.VectorSubcoreMesh(core_axis_name="c", subcore_axis_name="s")
    return pl.kernel(
        _sc_expand,
        mesh=mesh,
        out_type=jax.ShapeDtypeStruct((_NG, _M, _M), jnp.float32),
        scratch_types=[
            pltpu.VMEM((512, 128), jnp.int32),
            pltpu.VMEM((_BATCH, _M), jnp.float32),
        ],
        compiler_params=pltpu.CompilerParams(needs_layout_passes=False),
    )


def _sc_expand(xsp_hbm, out_hbm, plane_v, rows_v):
    g = lax.axis_index("s")
    h = lax.axis_index("c")
    pltpu.sync_copy(xsp_hbm.at[g], plane_v)
    nn = 512 - 32 * lax.rem(g, 8)
    base_row = h * 256
    jot = lax.iota(jnp.int32, 16)

    def batch_body(bidx, _):
        i0 = base_row + bidx * _BATCH
        for r in range(_BATCH):
            i = i0 + r
            offi = i * 511 - ((i * (i - 1)) >> 1)
            upper_c = offi - i - 1
            rowf = jnp.where(i < nn, 1.0, 0.0)

            def chunk_body(k, _):
                j = jot + k * 16
                offj = j * 511 - ((j * (j - 1)) >> 1)
                idx = jnp.where(j > i, upper_c + j, offj + (i - 1) - j)
                idx = jnp.maximum(idx, 0)
                w_idx = lax.shift_right_logical(idx, 1)
                w = plsc.load_gather(
                    plane_v, [lax.shift_right_logical(w_idx, 7), w_idx & 127])
                v = lax.shift_right_logical(w, (idx & 1) << 4) & 0xFFFF
                m = jnp.where((j < nn) & (j != i), rowf, 0.0)
                rows_v[r, pl.ds(k * 16, 16)] = v.astype(jnp.float32) * m
                return 0

            lax.fori_loop(0, 32, chunk_body, 0)
        pltpu.sync_copy(rows_v, out_hbm.at[g, pl.ds(i0, _BATCH)])
        return 0

    lax.fori_loop(0, 256 // _BATCH, batch_body, 0)


def _as_tiles(x):
    return (x.reshape(64, 1022, 128, 2)
             .transpose(0, 1, 3, 2)
             .reshape(64, 1022, 2, 1, 128))


def kernel(adj_logits, gumbel_noise):
    a5 = _as_tiles(adj_logits)
    b5 = _as_tiles(gumbel_noise)
    la, lb = _merge_matrices()
    la = jnp.asarray(la, dtype=jnp.bfloat16)
    lb = jnp.asarray(lb, dtype=jnp.bfloat16)
    pmat = jnp.asarray(_pack_matrix(), dtype=jnp.bfloat16)
    xsp = _phase1(a5, b5, la, lb, pmat)
    return _sc_expand_fn()(xsp.reshape(_NG, _M, 128))


# speedup vs baseline: 16.0628x; 1.0445x over previous
import functools

import jax
import jax.numpy as jnp
import numpy as np
from jax import lax
from jax.experimental import pallas as pl
from jax.experimental.pallas import tpu as pltpu
from jax.experimental.pallas import tpu_sc as plsc

_M = 512
_PAIRS = _M * (_M - 1) // 2
_RPC = 511
_BLK = 8 * _RPC
_NG = 16
_NC = 4


def _merge_matrices() -> tuple[np.ndarray, np.ndarray]:
    la = np.zeros((512, 1022), dtype=np.float32)
    lb = np.zeros((512, 1022), dtype=np.float32)
    r = np.arange(511)
    la[r, 2 * r] = 1.0
    lb[r, 2 * r + 1] = 1.0
    return la, lb


def _pack_matrix() -> np.ndarray:
    p = np.zeros((128, 64), dtype=np.float32)
    u = np.arange(64)
    p[2 * u, u] = 1.0
    p[2 * u + 1, u] = 65536.0
    return p


def _start_copies(a_hbm, b_hbm, bufs, sems, step, slot):
    for ch in range(2):
        pltpu.make_async_copy(
            a_hbm.at[pl.ds(step * 8, 8), :, ch, 0, :],
            bufs[ch].at[slot], sems.at[slot, ch]).start()
        pltpu.make_async_copy(
            b_hbm.at[pl.ds(step * 8, 8), :, ch, 0, :],
            bufs[2 + ch].at[slot], sems.at[slot, 2 + ch]).start()


def _wait_copies(a_hbm, b_hbm, bufs, sems, step, slot):
    for ch in range(2):
        pltpu.make_async_copy(
            a_hbm.at[pl.ds(step * 8, 8), :, ch, 0, :],
            bufs[ch].at[slot], sems.at[slot, ch]).wait()
        pltpu.make_async_copy(
            b_hbm.at[pl.ds(step * 8, 8), :, ch, 0, :],
            bufs[2 + ch].at[slot], sems.at[slot, 2 + ch]).wait()


def _phase1_body(a_hbm, b_hbm, la_ref, lb_ref, p_ref, o_ref,
                 a0b, a1b, b0b, b1b, sems):
    b = pl.program_id(0)
    slot = lax.rem(b, 2)
    bufs = (a0b, a1b, b0b, b1b)

    @pl.when(b == 0)
    def _():
        _start_copies(a_hbm, b_hbm, bufs, sems, 0, 0)

    @pl.when(b < 7)
    def _():
        _start_copies(a_hbm, b_hbm, bufs, sems, b + 1, 1 - slot)

    _wait_copies(a_hbm, b_hbm, bufs, sems, b, slot)

    s0 = a0b[slot] + b0b[slot]
    s1 = a1b[slot] + b1b[slot]
    x = (s0 >= s1).astype(jnp.bfloat16)
    for gamma in range(2):
        g4 = 4 * gamma
        xs = x[g4] + x[g4 + 1] + x[g4 + 2] + x[g4 + 3]
        ya = jnp.dot(la_ref[...], xs, preferred_element_type=jnp.float32)
        yb = jnp.dot(lb_ref[...], xs, preferred_element_type=jnp.float32)
        pa = jnp.dot(ya.astype(jnp.bfloat16), p_ref[...],
                     preferred_element_type=jnp.float32)
        pb = jnp.dot(yb.astype(jnp.bfloat16), p_ref[...],
                     preferred_element_type=jnp.float32)
        packed = jnp.concatenate([pa, pb], axis=1).astype(jnp.int32)
        o_ref[pl.ds(gamma * 512, 512), :] = packed


def _phase1(a5, b5, la, lb, pmat):
    return pl.pallas_call(
        _phase1_body,
        grid=(8,),
        in_specs=[
            pl.BlockSpec(memory_space=pl.ANY),
            pl.BlockSpec(memory_space=pl.ANY),
            pl.BlockSpec((512, 1022), lambda b: (0, 0)),
            pl.BlockSpec((512, 1022), lambda b: (0, 0)),
            pl.BlockSpec((128, 64), lambda b: (0, 0)),
        ],
        out_specs=pl.BlockSpec((1024, 128), lambda b: (b, 0)),
        out_shape=jax.ShapeDtypeStruct((8192, 128), jnp.int32),
        scratch_shapes=[
            pltpu.VMEM((2, 8, 1022, 128), jnp.float32),
            pltpu.VMEM((2, 8, 1022, 128), jnp.float32),
            pltpu.VMEM((2, 8, 1022, 128), jnp.float32),
            pltpu.VMEM((2, 8, 1022, 128), jnp.float32),
            pltpu.SemaphoreType.DMA((2, 4)),
        ],
    )(a5, b5, la, lb, pmat)


_BATCH = 8


@functools.cache
def _sc_expand_fn():
    mesh = plsc.VectorSubcoreMesh(core_axis_name="c", subcore_axis_name="s")
    return pl.kernel(
        _sc_expand,
        mesh=mesh,
        out_type=jax.ShapeDtypeStruct((_NG, _M, _M), jnp.float32),
        scratch_types=[
            pltpu.VMEM((512, 128), jnp.int32),
            pltpu.VMEM((2, _BATCH, _M), jnp.float32),
            pltpu.VMEM((_BATCH, _M), jnp.float32),
            pltpu.SemaphoreType.DMA,
        ],
        compiler_params=pltpu.CompilerParams(needs_layout_passes=False),
    )


def _sc_expand(xsp_hbm, out_hbm, plane_v, rows_v, zrow_v, dsem):
    g = lax.axis_index("s")
    h = lax.axis_index("c")
    pltpu.sync_copy(xsp_hbm.at[g], plane_v)
    nn = 512 - 32 * lax.rem(g, 8)
    base_row = h * 256
    jot = lax.iota(jnp.int32, 16)
    zero16 = jnp.zeros((16,), jnp.float32)
    nb = 256 // _BATCH

    def zinit(k, _):
        for r in range(_BATCH):
            zrow_v[r, pl.ds(k * 16, 16)] = zero16
        return 0

    lax.fori_loop(0, 32, zinit, 0)
    vb = jnp.clip((nn - base_row + _BATCH - 1) // _BATCH, 0, nb)

    def _drain():
        return pltpu.make_async_copy(
            rows_v.at[0], out_hbm.at[g, pl.ds(base_row, _BATCH)], dsem)

    def batch_body(bidx, _):
        i0 = base_row + bidx * _BATCH
        slot = lax.rem(bidx, 2)
        valid = i0 < nn

        @pl.when(valid)
        def _():
            @pl.when(bidx >= 2)
            def _():
                _drain().wait()

            ucs = []
            for r in range(_BATCH):
                i = i0 + r
                ucs.append(i * 511 - ((i * (i - 1)) >> 1) - i - 1)

            def kloop(k, _):
                j = jot + k * 16
                lowb = j * 511 - ((j * (j - 1)) >> 1) - j - 1
                mcol = jnp.where(j < nn, 1.0, 0.0)
                for r in range(_BATCH):
                    i = i0 + r
                    idx = jnp.where(j > i, ucs[r] + j, lowb + i)
                    idx = jnp.where(j == i, 131071, idx)
                    w_idx = lax.shift_right_logical(idx, 1)
                    w = plsc.load_gather(
                        plane_v,
                        [lax.shift_right_logical(w_idx, 7), w_idx & 127])
                    v = lax.shift_right_logical(w, (idx & 1) << 4) & 0xFFFF
                    rows_v[slot, r, pl.ds(k * 16, 16)] = (
                        v.astype(jnp.float32) * mcol)
                return 0

            lax.fori_loop(0, 32, kloop, 0)
            pltpu.make_async_copy(
                rows_v.at[slot], out_hbm.at[g, pl.ds(i0, _BATCH)],
                dsem).start()

        @pl.when(jnp.logical_not(valid))
        def _():
            pltpu.sync_copy(zrow_v, out_hbm.at[g, pl.ds(i0, _BATCH)])

        return 0

    lax.fori_loop(0, nb, batch_body, 0)

    @pl.when(vb >= 1)
    def _():
        _drain().wait()

    @pl.when(vb >= 2)
    def _():
        _drain().wait()


def _as_tiles(x):
    return (x.reshape(64, 1022, 128, 2)
             .transpose(0, 1, 3, 2)
             .reshape(64, 1022, 2, 1, 128))


def kernel(adj_logits, gumbel_noise):
    a5 = _as_tiles(adj_logits)
    b5 = _as_tiles(gumbel_noise)
    la, lb = _merge_matrices()
    la = jnp.asarray(la, dtype=jnp.bfloat16)
    lb = jnp.asarray(lb, dtype=jnp.bfloat16)
    pmat = jnp.asarray(_pack_matrix(), dtype=jnp.bfloat16)
    xsp = _phase1(a5, b5, la, lb, pmat)
    return _sc_expand_fn()(xsp.reshape(_NG, _M, 128))
